# Initial kernel scaffold; baseline (speedup 1.0000x reference)
import jax
import jax.numpy as jnp
from jax.experimental import pallas as pl


def kernel(x, edge_index, batch, W1a, b1a, g1a, t1a, W1b, b1b, g1b, t1b, W2a, b2a, g2a, t2a, W2b, b2b, g2b, t2b, Wc1, bc1, gc1, tc1, Wc2, bc2):
    raise NotImplementedError("write your pallas kernel here")


# speedup vs baseline: 4.6076x; 4.6076x over previous
import functools

import jax
import jax.numpy as jnp
from jax import lax
from jax.experimental import pallas as pl
from jax.experimental.pallas import tpu as pltpu
from jax.experimental.pallas import tpu_sc as plsc

_N = 10000
_G = 64
_NC = 2
_NS = 16
_K = 128
_NPAD = 10112


@functools.partial(jax.jit, static_argnames=("feat_dim", "chunks"))
def _sc_scatter(table, init, src3, dst3, *, feat_dim, chunks):
    rps = _NPAD // _NS

    def body(table_r, init_r, src_r, dst_r, out_r, srcv, dstv, rows, acc, sem):
        c = lax.axis_index("c")
        s = lax.axis_index("s")
        pltpu.sync_copy(init_r.at[c, pl.ds(s * rps, rps)],
                        acc.at[pl.ds(s * rps, rps)])
        plsc.subcore_barrier()
        wid = c * _NS + s
        pltpu.sync_copy(src_r.at[wid], srcv)
        pltpu.sync_copy(dst_r.at[wid], dstv)

        def chunk(j, carry):
            pltpu.async_copy(table_r.at[srcv.at[j]], rows, sem).wait()
            pltpu.sync_copy(rows, acc.at[dstv.at[j]], add=True)
            return carry

        lax.fori_loop(0, chunks, chunk, 0)
        plsc.subcore_barrier()
        pltpu.sync_copy(acc.at[pl.ds(s * rps, rps)],
                        out_r.at[c, pl.ds(s * rps, rps)])

    return pl.kernel(
        body,
        out_type=jax.ShapeDtypeStruct((_NC, _NPAD, feat_dim), jnp.float32),
        mesh=plsc.VectorSubcoreMesh(core_axis_name="c", subcore_axis_name="s"),
        scratch_types=[
            pltpu.VMEM((chunks, _K), jnp.int32),
            pltpu.VMEM((chunks, _K), jnp.int32),
            pltpu.VMEM((_K, feat_dim), jnp.float32),
            pltpu.VMEM_SHARED((_NPAD, feat_dim), jnp.float32),
            pltpu.SemaphoreType.DMA,
        ],
        compiler_params=pltpu.CompilerParams(use_tc_tiling_on_sc=False),
    )(table, init, src3, dst3)


def _dot(a, b):
    return lax.dot_general(a, b, (((1,), (0,)), ((), ())),
                           precision=lax.Precision.HIGHEST,
                           preferred_element_type=jnp.float32)


def _bn_masked(u, g, t, mask):
    m = jnp.sum(u * mask, axis=0, keepdims=True) * (1.0 / _N)
    d = (u - m) * mask
    v = jnp.sum(d * d, axis=0, keepdims=True) * (1.0 / _N)
    return (u - m) * lax.rsqrt(v + 1e-5) * g + t


def _mlp(z, Wa, ba, ga, ta, Wb, bb, gb, tb, mask):
    u = _bn_masked(_dot(z, Wa) + ba, ga, ta, mask)
    u = jnp.maximum(u, 0.0)
    u = _bn_masked(_dot(u, Wb) + bb, gb, tb, mask)
    return jnp.maximum(u, 0.0)


def _tc_layer_body(p_ref, Wa_ref, ba_ref, ga_ref, ta_ref,
                   Wb_ref, bb_ref, gb_ref, tb_ref, h_ref):
    z = p_ref[0] + p_ref[1]
    mask = (lax.broadcasted_iota(jnp.int32, (_NPAD, 1), 0)
            < _N).astype(jnp.float32)
    h_ref[...] = _mlp(z, Wa_ref[...], ba_ref[...], ga_ref[...], ta_ref[...],
                      Wb_ref[...], bb_ref[...], gb_ref[...], tb_ref[...],
                      mask)


def _tc_head_body(p_ref, batch_ref, Wa_ref, ba_ref, ga_ref, ta_ref,
                  Wb_ref, bb_ref, gb_ref, tb_ref,
                  Wc1_ref, bc1_ref, gc1_ref, tc1_ref, Wc2_ref, bc2_ref,
                  out_ref):
    z = p_ref[0] + p_ref[1]
    mask = (lax.broadcasted_iota(jnp.int32, (_NPAD, 1), 0)
            < _N).astype(jnp.float32)
    h = _mlp(z, Wa_ref[...], ba_ref[...], ga_ref[...], ta_ref[...],
             Wb_ref[...], bb_ref[...], gb_ref[...], tb_ref[...], mask)
    onehot = (batch_ref[...] ==
              lax.broadcasted_iota(jnp.int32, (_G, _NPAD), 0)
              ).astype(jnp.float32)
    sums = _dot(onehot, h)
    cnt = jnp.sum(onehot, axis=1, keepdims=True)
    hm = sums / jnp.maximum(cnt, 1.0)
    zc = _dot(hm, Wc1_ref[...]) + bc1_ref[...]
    m = jnp.mean(zc, axis=0, keepdims=True)
    v = jnp.mean((zc - m) ** 2, axis=0, keepdims=True)
    zc = (zc - m) * lax.rsqrt(v + 1e-5) * gc1_ref[...] + tc1_ref[...]
    zc = jnp.maximum(zc, 0.0)
    zc = _dot(zc, Wc2_ref[...]) + bc2_ref[...]
    mx = jnp.max(zc, axis=1, keepdims=True)
    out_ref[...] = (zc - mx) - jnp.log(
        jnp.sum(jnp.exp(zc - mx), axis=1, keepdims=True))


def kernel(x, edge_index, batch, W1a, b1a, g1a, t1a, W1b, b1b, g1b, t1b,
           W2a, b2a, g2a, t2a, W2b, b2b, g2b, t2b,
           Wc1, bc1, gc1, tc1, Wc2, bc2):
    F = x.shape[1]
    H = W1a.shape[1]
    E = edge_index.shape[1]
    W = _NC * _NS
    chunks = -(-E // (W * _K))
    epad = W * _K * chunks
    extra = epad - E

    src = jnp.concatenate([edge_index[0],
                           jnp.zeros((extra,), jnp.int32)]).reshape(W, chunks, _K)
    dst = jnp.concatenate([edge_index[1],
                           jnp.full((extra,), _N, jnp.int32)]).reshape(W, chunks, _K)

    row2 = lambda a: a.reshape(1, -1)
    zpadF = jnp.zeros((_NPAD, F), jnp.float32)
    xpad = jnp.concatenate([x, jnp.zeros((_NPAD - _N, F), jnp.float32)], axis=0)

    p1 = _sc_scatter(x, jnp.stack([xpad, zpadF]), src, dst,
                     feat_dim=F, chunks=chunks)

    h = pl.pallas_call(
        _tc_layer_body,
        out_shape=jax.ShapeDtypeStruct((_NPAD, H), jnp.float32),
    )(p1, W1a, row2(b1a), row2(g1a), row2(t1a),
      W1b, row2(b1b), row2(g1b), row2(t1b))

    zpadH = jnp.zeros((_NPAD, H), jnp.float32)
    p2 = _sc_scatter(h, jnp.stack([h, zpadH]), src, dst,
                     feat_dim=H, chunks=chunks)

    batchp = jnp.concatenate([batch,
                              jnp.full((_NPAD - _N,), _G, jnp.int32)]).reshape(1, -1)
    out = pl.pallas_call(
        _tc_head_body,
        out_shape=jax.ShapeDtypeStruct((_G, Wc2.shape[1]), jnp.float32),
    )(p2, batchp, W2a, row2(b2a), row2(g2a), row2(t2a),
      W2b, row2(b2b), row2(g2b), row2(t2b),
      Wc1, row2(bc1), row2(gc1), row2(tc1), Wc2, row2(bc2))
    return out
